# baseline (speedup 1.0000x reference)
import jax
import jax.numpy as jnp
from jax.experimental import pallas as pl


def _pass1_body(adj_ref, featfull_ref, featblk_ref, ax_ref, g_ref):
    a = adj_ref[...].astype(jnp.bfloat16)
    ax = jnp.dot(a, featfull_ref[...], preferred_element_type=jnp.float32)
    ax_ref[...] = ax
    g_ref[...] = (ax * featblk_ref[...]).astype(jnp.bfloat16)


def _pass2_body(adj_ref, gfull_ref, ax_ref, w1_ref, w2_ref, bias_ref,
                pre_ref, out_ref):
    a = adj_ref[...].astype(jnp.bfloat16)
    axx = jnp.dot(a, gfull_ref[...], preferred_element_type=jnp.float32)
    dn = (((1,), (1,)), ((), ()))
    pre = (
        jax.lax.dot_general(ax_ref[...], w1_ref[...], dn,
                            precision=jax.lax.Precision.HIGHEST,
                            preferred_element_type=jnp.float32)
        + jax.lax.dot_general(axx, w2_ref[...], dn,
                              precision=jax.lax.Precision.HIGHEST,
                              preferred_element_type=jnp.float32)
        + bias_ref[...]
    )
    pre_ref[...] = pre
    out_ref[...] = jnp.where(pre > 0, pre, jnp.exp(pre) - 1.0)


def kernel(features, adj, W1, b1, W2, b2):
    N, H = features.shape
    BI = 400
    R = N // BI
    feat16 = features.astype(jnp.bfloat16)
    bias = (b1 + b2).reshape(1, H)

    ax, g = pl.pallas_call(
        _pass1_body,
        grid=(R,),
        in_specs=[
            pl.BlockSpec((BI, N), lambda i: (i, 0)),
            pl.BlockSpec((N, H), lambda i: (0, 0)),
            pl.BlockSpec((BI, H), lambda i: (i, 0)),
        ],
        out_specs=[
            pl.BlockSpec((BI, H), lambda i: (i, 0)),
            pl.BlockSpec((BI, H), lambda i: (i, 0)),
        ],
        out_shape=[
            jax.ShapeDtypeStruct((N, H), jnp.float32),
            jax.ShapeDtypeStruct((N, H), jnp.bfloat16),
        ],
    )(adj, feat16, features)

    pre, out = pl.pallas_call(
        _pass2_body,
        grid=(R,),
        in_specs=[
            pl.BlockSpec((BI, N), lambda i: (i, 0)),
            pl.BlockSpec((N, H), lambda i: (0, 0)),
            pl.BlockSpec((BI, H), lambda i: (i, 0)),
            pl.BlockSpec((H, H), lambda i: (0, 0)),
            pl.BlockSpec((H, H), lambda i: (0, 0)),
            pl.BlockSpec((1, H), lambda i: (0, 0)),
        ],
        out_specs=[
            pl.BlockSpec((BI, H), lambda i: (i, 0)),
            pl.BlockSpec((BI, H), lambda i: (i, 0)),
        ],
        out_shape=[
            jax.ShapeDtypeStruct((N, H), jnp.float32),
            jax.ShapeDtypeStruct((N, H), jnp.float32),
        ],
    )(adj, g, ax, W1, W2, bias)
    return (pre, out)
